# Initial kernel scaffold; baseline (speedup 1.0000x reference)
import jax
import jax.numpy as jnp
from jax.experimental import pallas as pl


def kernel(train, test_items):
    raise NotImplementedError("write your pallas kernel here")


# speedup vs baseline: 3.3375x; 3.3375x over previous
import functools

import jax
import jax.numpy as jnp
from jax import lax
from jax.experimental import pallas as pl
from jax.experimental.pallas import tpu as pltpu
from jax.experimental.pallas import tpu_sc as plsc

N_USERS = 1024
N_ITEMS = 100000
N_TEST = 200

COL_BLK = 2048

_SC_INFO = plsc.get_sparse_core_info()
_NC = _SC_INFO.num_cores
_NS = _SC_INFO.num_subcores
_NW = _NC * _NS
_L = _SC_INFO.num_lanes

_TOTAL_IDX = N_USERS * N_TEST
_IDX_PER_W = _TOTAL_IDX // _NW


def _sum_body(train_ref, out_ref):
    out_ref[...] = jnp.sum(train_ref[...], axis=0, keepdims=True)


def _popularity_sum(train):
    grid = pl.cdiv(N_ITEMS, COL_BLK)
    score2d = pl.pallas_call(
        _sum_body,
        grid=(grid,),
        in_specs=[pl.BlockSpec((N_USERS, COL_BLK), lambda j: (0, j))],
        out_specs=pl.BlockSpec((1, COL_BLK), lambda j: (0, j)),
        out_shape=jax.ShapeDtypeStruct((1, N_ITEMS), jnp.float32),
    )(train)
    return score2d.reshape(N_ITEMS)


@functools.partial(
    pl.kernel,
    out_type=jax.ShapeDtypeStruct((_TOTAL_IDX,), jnp.float32),
    mesh=plsc.VectorSubcoreMesh(core_axis_name="c", subcore_axis_name="s"),
    scratch_types=[
        pltpu.VMEM((_IDX_PER_W,), jnp.int32),
        pltpu.VMEM((_IDX_PER_W,), jnp.float32),
        pltpu.SemaphoreType.DMA,
    ],
)
def _gather_kernel(score_hbm, idx_hbm, out_hbm, idx_v, out_v, sem):
    wid = lax.axis_index("s") * _NC + lax.axis_index("c")
    base = wid * _IDX_PER_W
    pltpu.sync_copy(idx_hbm.at[pl.ds(base, _IDX_PER_W)], idx_v)
    pltpu.async_copy(score_hbm.at[idx_v], out_v, sem).wait()
    pltpu.sync_copy(out_v, out_hbm.at[pl.ds(base, _IDX_PER_W)])


def kernel(train, test_items):
    score = _popularity_sum(train)
    idx = test_items.reshape(-1).astype(jnp.int32)
    flat = _gather_kernel(score, idx)
    return flat.reshape(N_USERS, N_TEST)
